# baseline (speedup 1.0000x reference)
import functools

import jax
import jax.numpy as jnp
from jax import lax
from jax.experimental import pallas as pl
from jax.experimental.pallas import tpu as pltpu
from jax.experimental.pallas import tpu_sc as plsc

N = 10000
E = 320000
NC, NS = 2, 16
NW = NC * NS
K = 128
EPT = 10240
NB = EPT // K
EPAD = EPT * NW
NPAD = 10112
NPT = NPAD // NS
R = 2528
M = 8
P = 4

_BN = 1.0 / (1.001 ** 0.5)
_f32 = jnp.float32


def _make_agg(F):
    @functools.partial(
        pl.kernel,
        out_type=jax.ShapeDtypeStruct((NC, NPAD, F), _f32),
        mesh=plsc.VectorSubcoreMesh(core_axis_name="c", subcore_axis_name="s"),
        scratch_types=[
            pltpu.VMEM((NB, K), jnp.int32),
            pltpu.VMEM((NB, K), jnp.int32),
            pltpu.VMEM((M, K, F), _f32),
            pltpu.VMEM_SHARED((NPAD, F), _f32),
            pltpu.SemaphoreType.DMA((M,)),
            pltpu.SemaphoreType.DMA((M,)),
        ],
        compiler_params=pltpu.CompilerParams(use_tc_tiling_on_sc=False),
    )
    def agg(g_hbm, src_hbm, dst_hbm, zero_hbm, out_hbm, src_v, dst_v, buf,
            acc, gsem, ssem):
        cid = lax.axis_index("c")
        sid = lax.axis_index("s")
        wid = cid * NS + sid
        pltpu.sync_copy(src_hbm.at[wid], src_v)
        pltpu.sync_copy(dst_hbm.at[wid], dst_v)

        @pl.when(cid == 0)
        def _():
            pltpu.sync_copy(g_hbm.at[pl.ds(sid * NPT, NPT)],
                            acc.at[pl.ds(sid * NPT, NPT)])

        @pl.when(cid == 1)
        def _():
            pltpu.sync_copy(zero_hbm, acc.at[pl.ds(sid * NPT, NPT)])

        plsc.subcore_barrier()

        for b in range(P):
            pltpu.async_copy(g_hbm.at[src_v.at[b]], buf.at[b], gsem.at[b])

        @pl.loop(0, NB, step=M)
        def _(j):
            for b in range(M):
                t = j + b
                pltpu.make_async_copy(
                    g_hbm.at[src_v.at[t]], buf.at[b], gsem.at[b]).wait()
                pltpu.async_copy(buf.at[b], acc.at[dst_v.at[t]], ssem.at[b],
                                 add=True)
                bg = (b + P) % M

                def _prefetch(bg=bg, t=t):
                    pltpu.make_async_copy(
                        buf.at[bg], acc.at[dst_v.at[t]], ssem.at[bg]).wait()
                    pltpu.async_copy(g_hbm.at[src_v.at[t + P]], buf.at[bg],
                                     gsem.at[bg])

                def _prefetch_first(bg=bg, t=t):
                    pltpu.async_copy(g_hbm.at[src_v.at[t + P]], buf.at[bg],
                                     gsem.at[bg])

                if b + P < M:
                    pl.when(j > 0)(_prefetch)
                    pl.when(jnp.logical_and(j == 0, t + P < NB))(
                        _prefetch_first)
                else:
                    pl.when(t + P < NB)(_prefetch)

        for b in range(M):
            pltpu.make_async_copy(
                buf.at[b], acc.at[dst_v.at[0]], ssem.at[b]).wait()
        plsc.subcore_barrier()
        pltpu.sync_copy(acc.at[pl.ds(sid * NPT, NPT)],
                        out_hbm.at[cid, pl.ds(sid * NPT, NPT)])

    return agg


def _elu(h):
    return jnp.where(h > 0, h, jnp.exp(jnp.minimum(h, 0.0)) - 1.0)


def _rows(shape):
    if len(shape) == 2:
        return pl.BlockSpec((R, shape[1]), lambda i: (i, 0))
    return pl.BlockSpec((shape[0], R, shape[2]), lambda i: (0, i, 0))


def _rep(shape):
    nd = len(shape)
    return pl.BlockSpec(shape, lambda i: (0,) * nd)


def _enc_body(x_ref, pdeg_ref, w1, b1, ga1, be1, w2, b2, ga2, be2,
              wc1, feat_o, g1_o, dinv_o):
    deg = pdeg_ref[0] + pdeg_ref[1]
    dinv = lax.rsqrt(jnp.maximum(deg, 1e-12))
    h = jnp.dot(x_ref[...], w1[...], preferred_element_type=_f32) + b1[...]
    h = _elu(h * (ga1[...] * _BN) + be1[...])
    h = jnp.dot(h, w2[...], preferred_element_type=_f32) + b2[...]
    feat = _elu(h * (ga2[...] * _BN) + be2[...])
    feat_o[...] = feat
    g1_o[...] = jnp.dot(feat, wc1[...], preferred_element_type=_f32) * dinv
    dinv_o[...] = dinv


def _mid_body(p_ref, d_ref, b_ref, w_ref, k_ref, gout_o):
    dp = jnp.dot(d_ref[...], k_ref[...], preferred_element_type=_f32)
    h = jnp.maximum((p_ref[0] + p_ref[1]) * dp + b_ref[...], 0.0)
    gout_o[...] = jnp.dot(h, w_ref[...], preferred_element_type=_f32) * dp


def _midp(p, dinv2, b, w):
    fi = w.shape[0]
    rep = 128 // fi
    rows = NPAD * fi // 128
    pp = p.reshape(NC, rows, 128)
    dr = dinv2.reshape(rows, rep)
    w_ = jnp.kron(jnp.eye(rep, dtype=_f32), w)
    k_ = jnp.kron(jnp.eye(rep, dtype=_f32), jnp.ones((1, fi), _f32))
    b_ = jnp.tile(b, rep).reshape(1, 128)
    rp = NPAD // 16
    return pl.pallas_call(
        _mid_body,
        grid=(rows // rp,),
        in_specs=[pl.BlockSpec((NC, rp, 128), lambda i: (0, i, 0)),
                  pl.BlockSpec((rp, rep), lambda i: (i, 0)),
                  _rep((1, 128)), _rep((128, 128)), _rep((rep, 128))],
        out_specs=pl.BlockSpec((rp, 128), lambda i: (i, 0)),
        out_shape=jax.ShapeDtypeStruct((rows, 128), _f32),
    )(pp, dr, b_, w_, k_)


def _mid4_body(p_ref, d4_ref, d8_ref, b_ref, w_ref, k4_ref, k8_ref, gm_o):
    dp = jnp.dot(d4_ref[...], k4_ref[...], preferred_element_type=_f32)
    h = jnp.maximum((p_ref[0] + p_ref[1]) * dp + b_ref[...], 0.0)
    h3 = h.reshape(h.shape[0] // 2, 2, 128)
    w4 = w_ref[...]
    gm = jnp.concatenate(
        [jnp.dot(h3[:, 0, :], w4, preferred_element_type=_f32),
         jnp.dot(h3[:, 1, :], w4, preferred_element_type=_f32)], axis=-1)
    dp16 = jnp.dot(d8_ref[...], k8_ref[...], preferred_element_type=_f32)
    gm_o[...] = gm * dp16


def _mid4(p, dinv2, b, w):
    pp = p.reshape(NC, NPAD // 4, 128)
    d4 = dinv2.reshape(NPAD // 4, 4)
    d8 = dinv2.reshape(NPAD // 8, 8)
    w4 = jnp.kron(jnp.eye(4, dtype=_f32), w)
    k4 = jnp.kron(jnp.eye(4, dtype=_f32), jnp.ones((1, 32), _f32))
    k8 = jnp.kron(jnp.eye(8, dtype=_f32), jnp.ones((1, 16), _f32))
    b4 = jnp.tile(b, 4).reshape(1, 128)
    rp = NPAD // 8
    return pl.pallas_call(
        _mid4_body,
        grid=(2,),
        in_specs=[pl.BlockSpec((NC, rp, 128), lambda i: (0, i, 0)),
                  pl.BlockSpec((rp, 4), lambda i: (i, 0)),
                  pl.BlockSpec((rp // 2, 8), lambda i: (i, 0)),
                  _rep((1, 128)), _rep((128, 64)), _rep((4, 128)),
                  _rep((8, 128))],
        out_specs=pl.BlockSpec((rp // 2, 128), lambda i: (i, 0)),
        out_shape=jax.ShapeDtypeStruct((NPAD // 8, 128), _f32),
    )(pp, d4, d8, b4, w4, k4, k8)


def _fin_body(p_ref, dinv_ref, b_ref, feat_ref,
              wd1, bd1, ga1, be1, wd2, bd2, ga2, be2, wd3, bd3, ga3, be3,
              ct_ref, z_o, mu_o, lv_o, de_o, q_o):
    dinv = dinv_ref[...]
    h = jnp.maximum((p_ref[0] + p_ref[1]) * dinv + b_ref[...], 0.0)
    mu = h[:, :8]
    lv = h[:, 8:]
    z = jnp.concatenate([feat_ref[...], mu], axis=1)
    d = jnp.dot(z, wd1[...], preferred_element_type=_f32) + bd1[...]
    d = _elu(d * (ga1[...] * _BN) + be1[...])
    d = jnp.dot(d, wd2[...], preferred_element_type=_f32) + bd2[...]
    d = _elu(d * (ga2[...] * _BN) + be2[...])
    d = jnp.dot(d, wd3[...], preferred_element_type=_f32) + bd3[...]
    d = d * (ga3[...] * _BN) + be3[...]
    de = 1.0 / (1.0 + jnp.exp(-d))
    ct = ct_ref[...]
    d2 = (jnp.sum(z * z, axis=1, keepdims=True)
          - 2.0 * jnp.dot(z, ct, preferred_element_type=_f32)
          + jnp.sum(ct * ct, axis=0, keepdims=True))
    qu = 1.0 / ((1.0 + d2 * 1.25) + 1e-8)
    qu = jnp.exp(0.9 * jnp.log(qu))
    z_o[...] = z
    mu_o[...] = mu
    lv_o[...] = lv
    de_o[...] = de
    q_o[...] = qu / jnp.sum(qu, axis=1, keepdims=True)


def kernel(x, edge_index, params):
    pe, pc = params["enc"], params["conv"]
    pm, pv, pd = params["mean"], params["logvar"], params["dec"]

    padi = N + jnp.arange(EPAD - E, dtype=jnp.int32) % (NPAD - N)
    srcp = jnp.concatenate([edge_index[0], padi]).reshape(NW, NB, K)
    dstp = jnp.concatenate([edge_index[1], padi]).reshape(NW, NB, K)
    xpad = jnp.concatenate([x, jnp.zeros((NPAD - N, x.shape[1]), _f32)], axis=0)
    ones32 = jnp.ones((NPAD, 32), _f32)
    wml1 = jnp.concatenate([pm[0]["W"], pv[0]["W"]], axis=1)
    z8 = jnp.zeros((8, 8), _f32)
    wblk = jnp.concatenate(
        [jnp.concatenate([pm[1]["W"], z8], axis=1),
         jnp.concatenate([z8, pv[1]["W"]], axis=1)], axis=0)
    bml1 = jnp.concatenate([pm[0]["b"], pv[0]["b"]])
    bml2 = jnp.concatenate([pm[1]["b"], pv[1]["b"]])
    ct = params["cluster"].T

    agg32 = _make_agg(32)
    agg16 = _make_agg(16)

    zero32 = jnp.zeros((NPT, 32), _f32)
    zero16 = jnp.zeros((NPT, 16), _f32)

    pdeg = agg32(ones32, dstp, dstp, zero32)
    pdeg1 = pdeg[:, :, :1]

    feat, g1, dinv2 = pl.pallas_call(
        _enc_body,
        grid=(NPAD // R,),
        in_specs=[_rows((NPAD, 128)), _rows((NC, NPAD, 1)),
                  _rep((128, 50)), _rep((1, 50)), _rep((1, 50)), _rep((1, 50)),
                  _rep((50, 20)), _rep((1, 20)), _rep((1, 20)), _rep((1, 20)),
                  _rep((20, 32))],
        out_specs=[_rows((NPAD, 20)), _rows((NPAD, 32)), _rows((NPAD, 1))],
        out_shape=[jax.ShapeDtypeStruct((NPAD, 20), _f32),
                   jax.ShapeDtypeStruct((NPAD, 32), _f32),
                   jax.ShapeDtypeStruct((NPAD, 1), _f32)],
    )(xpad, pdeg1,
      pe[0]["W"], pe[0]["b"].reshape(1, 50), pe[0]["g"].reshape(1, 50),
      pe[0]["be"].reshape(1, 50),
      pe[1]["W"], pe[1]["b"].reshape(1, 20), pe[1]["g"].reshape(1, 20),
      pe[1]["be"].reshape(1, 20),
      pc[0]["W"])

    p1 = agg32(g1, srcp, dstp, zero32)
    g2p = _midp(p1, dinv2, pc[0]["b"], pc[1]["W"])
    p2 = agg32(g2p.reshape(NPAD, 32), srcp, dstp, zero32)
    g3p = _midp(p2, dinv2, pc[1]["b"], pc[2]["W"])
    p3 = agg32(g3p.reshape(NPAD, 32), srcp, dstp, zero32)
    g4p = _midp(p3, dinv2, pc[2]["b"], pc[3]["W"])
    p4 = agg32(g4p.reshape(NPAD, 32), srcp, dstp, zero32)
    gmp = _mid4(p4, dinv2, pc[3]["b"], wml1)
    p5 = agg16(gmp.reshape(NPAD, 16), srcp, dstp, zero16)
    gm2p = _midp(p5, dinv2, bml1, wblk)
    p6 = agg16(gm2p.reshape(NPAD, 16), srcp, dstp, zero16)

    z, mu, lv, de, q = pl.pallas_call(
        _fin_body,
        grid=(NPAD // R,),
        in_specs=[_rows((NC, NPAD, 16)), _rows((NPAD, 1)),
                  _rep((1, 16)), _rows((NPAD, 20)),
                  _rep((28, 50)), _rep((1, 50)), _rep((1, 50)), _rep((1, 50)),
                  _rep((50, 60)), _rep((1, 60)), _rep((1, 60)), _rep((1, 60)),
                  _rep((60, 128)), _rep((1, 128)), _rep((1, 128)), _rep((1, 128)),
                  _rep((28, 20))],
        out_specs=[_rows((N, 28)), _rows((N, 8)), _rows((N, 8)),
                   _rows((N, 128)), _rows((N, 20))],
        out_shape=[jax.ShapeDtypeStruct((N, 28), _f32),
                   jax.ShapeDtypeStruct((N, 8), _f32),
                   jax.ShapeDtypeStruct((N, 8), _f32),
                   jax.ShapeDtypeStruct((N, 128), _f32),
                   jax.ShapeDtypeStruct((N, 20), _f32)],
    )(p6, dinv2, bml2.reshape(1, 16), feat,
      pd[0]["W"], pd[0]["b"].reshape(1, 50), pd[0]["g"].reshape(1, 50),
      pd[0]["be"].reshape(1, 50),
      pd[1]["W"], pd[1]["b"].reshape(1, 60), pd[1]["g"].reshape(1, 60),
      pd[1]["be"].reshape(1, 60),
      pd[2]["W"], pd[2]["b"].reshape(1, 128), pd[2]["g"].reshape(1, 128),
      pd[2]["be"].reshape(1, 128),
      ct)

    return z, mu, lv, de, q


# speedup vs baseline: 1.0088x; 1.0088x over previous
import functools

import jax
import jax.numpy as jnp
import numpy as np
from jax import lax
from jax.experimental import pallas as pl
from jax.experimental.pallas import tpu as pltpu
from jax.experimental.pallas import tpu_sc as plsc

N = 10000
E = 320000
NC, NS = 2, 16
NW = NC * NS
K = 128
EPT = 10240
NB = EPT // K
EPAD = EPT * NW
NPAD = 10112
NPT = NPAD // NS
R = 2528
M = 8
P = 4

_BN = 1.0 / (1.001 ** 0.5)
_f32 = jnp.float32


def _make_agg(F, col_out=False):
    @functools.partial(
        pl.kernel,
        out_type=jax.ShapeDtypeStruct((NC, NPAD, 8 if col_out else F), _f32),
        mesh=plsc.VectorSubcoreMesh(core_axis_name="c", subcore_axis_name="s"),
        scratch_types=[
            pltpu.VMEM((NB, K), jnp.int32),
            pltpu.VMEM((NB, K), jnp.int32),
            pltpu.VMEM((M, K, F), _f32),
            pltpu.VMEM_SHARED((NPAD, F), _f32),
            pltpu.SemaphoreType.DMA((M,)),
            pltpu.SemaphoreType.DMA((M,)),
        ],
        compiler_params=pltpu.CompilerParams(use_tc_tiling_on_sc=False),
    )
    def agg(g_hbm, src_hbm, dst_hbm, zero_hbm, out_hbm, src_v, dst_v, buf,
            acc, gsem, ssem):
        cid = lax.axis_index("c")
        sid = lax.axis_index("s")
        wid = cid * NS + sid
        pltpu.sync_copy(src_hbm.at[wid], src_v)
        pltpu.sync_copy(dst_hbm.at[wid], dst_v)

        @pl.when(cid == 0)
        def _():
            pltpu.sync_copy(g_hbm.at[pl.ds(sid * NPT, NPT)],
                            acc.at[pl.ds(sid * NPT, NPT)])

        @pl.when(cid == 1)
        def _():
            pltpu.sync_copy(zero_hbm, acc.at[pl.ds(sid * NPT, NPT)])

        plsc.subcore_barrier()

        for b in range(P):
            pltpu.async_copy(g_hbm.at[src_v.at[b]], buf.at[b], gsem.at[b])

        @pl.loop(0, NB, step=M)
        def _(j):
            for b in range(M):
                t = j + b
                pltpu.make_async_copy(
                    g_hbm.at[src_v.at[t]], buf.at[b], gsem.at[b]).wait()
                pltpu.async_copy(buf.at[b], acc.at[dst_v.at[t]], ssem.at[b],
                                 add=True)
                bg = (b + P) % M

                def _prefetch(bg=bg, t=t):
                    pltpu.make_async_copy(
                        buf.at[bg], acc.at[dst_v.at[t]], ssem.at[bg]).wait()
                    pltpu.async_copy(g_hbm.at[src_v.at[t + P]], buf.at[bg],
                                     gsem.at[bg])

                def _prefetch_first(bg=bg, t=t):
                    pltpu.async_copy(g_hbm.at[src_v.at[t + P]], buf.at[bg],
                                     gsem.at[bg])

                if b + P < M:
                    pl.when(j > 0)(_prefetch)
                    pl.when(jnp.logical_and(j == 0, t + P < NB))(
                        _prefetch_first)
                else:
                    pl.when(t + P < NB)(_prefetch)

        for b in range(M):
            pltpu.make_async_copy(
                buf.at[b], acc.at[dst_v.at[0]], ssem.at[b]).wait()
        plsc.subcore_barrier()
        if col_out:
            pltpu.sync_copy(acc.at[pl.ds(sid * NPT, NPT), pl.ds(0, 8)],
                            out_hbm.at[cid, pl.ds(sid * NPT, NPT)])
        else:
            pltpu.sync_copy(acc.at[pl.ds(sid * NPT, NPT)],
                            out_hbm.at[cid, pl.ds(sid * NPT, NPT)])

    return agg


def _elu(h):
    return jnp.where(h > 0, h, jnp.exp(jnp.minimum(h, 0.0)) - 1.0)


def _rows(shape):
    if len(shape) == 2:
        return pl.BlockSpec((R, shape[1]), lambda i: (i, 0))
    return pl.BlockSpec((shape[0], R, shape[2]), lambda i: (0, i, 0))


def _rep(shape):
    nd = len(shape)
    return pl.BlockSpec(shape, lambda i: (0,) * nd)


def _enc_body(x_ref, pdeg_ref, w1, b1, ga1, be1, w2, b2, ga2, be2,
              wc1, feat_o, g1_o, dinv_o):
    deg = pdeg_ref[0] + pdeg_ref[1]
    dinv = lax.rsqrt(jnp.maximum(deg, 1e-12))
    h = jnp.dot(x_ref[...], w1[...], preferred_element_type=_f32) + b1[...]
    h = _elu(h * (ga1[...] * _BN) + be1[...])
    h = jnp.dot(h, w2[...], preferred_element_type=_f32) + b2[...]
    feat = _elu(h * (ga2[...] * _BN) + be2[...])
    feat_o[...] = feat
    g1_o[...] = jnp.dot(feat, wc1[...], preferred_element_type=_f32) * dinv
    dinv_o[...] = dinv


def _mid_body(p_ref, d_ref, b_ref, w_ref, k_ref, gout_o):
    dp = jnp.dot(d_ref[...], k_ref[...], preferred_element_type=_f32)
    h = jnp.maximum((p_ref[0] + p_ref[1]) * dp + b_ref[...], 0.0)
    gout_o[...] = jnp.dot(h, w_ref[...], preferred_element_type=_f32) * dp


def _midp(p, dinv2, b, w):
    fi = w.shape[0]
    rep = 128 // fi
    rows = NPAD * fi // 128
    pp = p.reshape(NC, rows, 128)
    dr = dinv2.reshape(rows, rep)
    w_ = jnp.kron(jnp.eye(rep, dtype=_f32), w)
    k_ = jnp.kron(jnp.eye(rep, dtype=_f32), jnp.ones((1, fi), _f32))
    b_ = jnp.tile(b, rep).reshape(1, 128)
    rp = NPAD // 16
    return pl.pallas_call(
        _mid_body,
        grid=(rows // rp,),
        in_specs=[pl.BlockSpec((NC, rp, 128), lambda i: (0, i, 0)),
                  pl.BlockSpec((rp, rep), lambda i: (i, 0)),
                  _rep((1, 128)), _rep((128, 128)), _rep((rep, 128))],
        out_specs=pl.BlockSpec((rp, 128), lambda i: (i, 0)),
        out_shape=jax.ShapeDtypeStruct((rows, 128), _f32),
    )(pp, dr, b_, w_, k_)


def _mid4_body(p_ref, d4_ref, d8_ref, b_ref, w_ref, k4_ref, k8_ref, gm_o):
    dp = jnp.dot(d4_ref[...], k4_ref[...], preferred_element_type=_f32)
    h = jnp.maximum((p_ref[0] + p_ref[1]) * dp + b_ref[...], 0.0)
    h3 = h.reshape(h.shape[0] // 2, 2, 128)
    w4 = w_ref[...]
    gm = jnp.concatenate(
        [jnp.dot(h3[:, 0, :], w4, preferred_element_type=_f32),
         jnp.dot(h3[:, 1, :], w4, preferred_element_type=_f32)], axis=-1)
    dp16 = jnp.dot(d8_ref[...], k8_ref[...], preferred_element_type=_f32)
    gm_o[...] = gm * dp16


def _mid4(p, dinv2, b, w):
    pp = p.reshape(NC, NPAD // 4, 128)
    d4 = dinv2.reshape(NPAD // 4, 4)
    d8 = dinv2.reshape(NPAD // 8, 8)
    w4 = jnp.kron(jnp.eye(4, dtype=_f32), w)
    k4 = jnp.kron(jnp.eye(4, dtype=_f32), jnp.ones((1, 32), _f32))
    k8 = jnp.kron(jnp.eye(8, dtype=_f32), jnp.ones((1, 16), _f32))
    b4 = jnp.tile(b, 4).reshape(1, 128)
    rp = NPAD // 8
    return pl.pallas_call(
        _mid4_body,
        grid=(2,),
        in_specs=[pl.BlockSpec((NC, rp, 128), lambda i: (0, i, 0)),
                  pl.BlockSpec((rp, 4), lambda i: (i, 0)),
                  pl.BlockSpec((rp // 2, 8), lambda i: (i, 0)),
                  _rep((1, 128)), _rep((128, 64)), _rep((4, 128)),
                  _rep((8, 128))],
        out_specs=pl.BlockSpec((rp // 2, 128), lambda i: (i, 0)),
        out_shape=jax.ShapeDtypeStruct((NPAD // 8, 128), _f32),
    )(pp, d4, d8, b4, w4, k4, k8)


def _fin_body(p_ref, dinv_ref, b_ref, feat_ref,
              wd1, bd1, ga1, be1, wd2, bd2, ga2, be2, wd3, bd3, ga3, be3,
              ct_ref, z_o, mu_o, lv_o, de_o, q_o):
    dinv = dinv_ref[...]
    h = jnp.maximum((p_ref[0] + p_ref[1]) * dinv + b_ref[...], 0.0)
    mu = h[:, :8]
    lv = h[:, 8:]
    z = jnp.concatenate([feat_ref[...], mu], axis=1)
    d = jnp.dot(z, wd1[...], preferred_element_type=_f32) + bd1[...]
    d = _elu(d * (ga1[...] * _BN) + be1[...])
    d = jnp.dot(d, wd2[...], preferred_element_type=_f32) + bd2[...]
    d = _elu(d * (ga2[...] * _BN) + be2[...])
    d = jnp.dot(d, wd3[...], preferred_element_type=_f32) + bd3[...]
    d = d * (ga3[...] * _BN) + be3[...]
    de = 1.0 / (1.0 + jnp.exp(-d))
    ct = ct_ref[...]
    d2 = (jnp.sum(z * z, axis=1, keepdims=True)
          - 2.0 * jnp.dot(z, ct, preferred_element_type=_f32)
          + jnp.sum(ct * ct, axis=0, keepdims=True))
    qu = 1.0 / ((1.0 + d2 * 1.25) + 1e-8)
    qu = jnp.exp(0.9 * jnp.log(qu))
    z_o[...] = z
    mu_o[...] = mu
    lv_o[...] = lv
    de_o[...] = de
    q_o[...] = qu / jnp.sum(qu, axis=1, keepdims=True)


def kernel(x, edge_index, params):
    pe, pc = params["enc"], params["conv"]
    pm, pv, pd = params["mean"], params["logvar"], params["dec"]

    padi = jnp.asarray(N + np.arange(EPAD - E) % (NPAD - N), jnp.int32)
    srcp = jnp.concatenate([edge_index[0], padi]).reshape(NW, NB, K)
    dstp = jnp.concatenate([edge_index[1], padi]).reshape(NW, NB, K)
    xpad = jnp.concatenate([x, jnp.zeros((NPAD - N, x.shape[1]), _f32)], axis=0)
    ones16 = jnp.ones((NPAD, 16), _f32)
    wml1 = jnp.concatenate([pm[0]["W"], pv[0]["W"]], axis=1)
    z8 = jnp.zeros((8, 8), _f32)
    wblk = jnp.concatenate(
        [jnp.concatenate([pm[1]["W"], z8], axis=1),
         jnp.concatenate([z8, pv[1]["W"]], axis=1)], axis=0)
    bml1 = jnp.concatenate([pm[0]["b"], pv[0]["b"]])
    bml2 = jnp.concatenate([pm[1]["b"], pv[1]["b"]])
    ct = params["cluster"].T

    agg32 = _make_agg(32)
    agg16 = _make_agg(16)
    aggdeg = _make_agg(16, col_out=True)

    zero32 = jnp.zeros((NPT, 32), _f32)
    zero16 = jnp.zeros((NPT, 16), _f32)

    pdeg1 = aggdeg(ones16, dstp, dstp, zero16)[:, :, :1]

    feat, g1, dinv2 = pl.pallas_call(
        _enc_body,
        grid=(NPAD // R,),
        in_specs=[_rows((NPAD, 128)), _rows((NC, NPAD, 1)),
                  _rep((128, 50)), _rep((1, 50)), _rep((1, 50)), _rep((1, 50)),
                  _rep((50, 20)), _rep((1, 20)), _rep((1, 20)), _rep((1, 20)),
                  _rep((20, 32))],
        out_specs=[_rows((NPAD, 20)), _rows((NPAD, 32)), _rows((NPAD, 1))],
        out_shape=[jax.ShapeDtypeStruct((NPAD, 20), _f32),
                   jax.ShapeDtypeStruct((NPAD, 32), _f32),
                   jax.ShapeDtypeStruct((NPAD, 1), _f32)],
    )(xpad, pdeg1,
      pe[0]["W"], pe[0]["b"].reshape(1, 50), pe[0]["g"].reshape(1, 50),
      pe[0]["be"].reshape(1, 50),
      pe[1]["W"], pe[1]["b"].reshape(1, 20), pe[1]["g"].reshape(1, 20),
      pe[1]["be"].reshape(1, 20),
      pc[0]["W"])

    p1 = agg32(g1, srcp, dstp, zero32)
    g2p = _midp(p1, dinv2, pc[0]["b"], pc[1]["W"])
    p2 = agg32(g2p.reshape(NPAD, 32), srcp, dstp, zero32)
    g3p = _midp(p2, dinv2, pc[1]["b"], pc[2]["W"])
    p3 = agg32(g3p.reshape(NPAD, 32), srcp, dstp, zero32)
    g4p = _midp(p3, dinv2, pc[2]["b"], pc[3]["W"])
    p4 = agg32(g4p.reshape(NPAD, 32), srcp, dstp, zero32)
    gmp = _mid4(p4, dinv2, pc[3]["b"], wml1)
    p5 = agg16(gmp.reshape(NPAD, 16), srcp, dstp, zero16)
    gm2p = _midp(p5, dinv2, bml1, wblk)
    p6 = agg16(gm2p.reshape(NPAD, 16), srcp, dstp, zero16)

    z, mu, lv, de, q = pl.pallas_call(
        _fin_body,
        grid=(NPAD // R,),
        in_specs=[_rows((NC, NPAD, 16)), _rows((NPAD, 1)),
                  _rep((1, 16)), _rows((NPAD, 20)),
                  _rep((28, 50)), _rep((1, 50)), _rep((1, 50)), _rep((1, 50)),
                  _rep((50, 60)), _rep((1, 60)), _rep((1, 60)), _rep((1, 60)),
                  _rep((60, 128)), _rep((1, 128)), _rep((1, 128)), _rep((1, 128)),
                  _rep((28, 20))],
        out_specs=[_rows((N, 28)), _rows((N, 8)), _rows((N, 8)),
                   _rows((N, 128)), _rows((N, 20))],
        out_shape=[jax.ShapeDtypeStruct((N, 28), _f32),
                   jax.ShapeDtypeStruct((N, 8), _f32),
                   jax.ShapeDtypeStruct((N, 8), _f32),
                   jax.ShapeDtypeStruct((N, 128), _f32),
                   jax.ShapeDtypeStruct((N, 20), _f32)],
    )(p6, dinv2, bml2.reshape(1, 16), feat,
      pd[0]["W"], pd[0]["b"].reshape(1, 50), pd[0]["g"].reshape(1, 50),
      pd[0]["be"].reshape(1, 50),
      pd[1]["W"], pd[1]["b"].reshape(1, 60), pd[1]["g"].reshape(1, 60),
      pd[1]["be"].reshape(1, 60),
      pd[2]["W"], pd[2]["b"].reshape(1, 128), pd[2]["g"].reshape(1, 128),
      pd[2]["be"].reshape(1, 128),
      ct)

    return z, mu, lv, de, q


# speedup vs baseline: 1.0490x; 1.0399x over previous
import functools

import jax
import jax.numpy as jnp
import numpy as np
from jax import lax
from jax.experimental import pallas as pl
from jax.experimental.pallas import tpu as pltpu
from jax.experimental.pallas import tpu_sc as plsc

N = 10000
E = 320000
NC, NS = 2, 16
NW = NC * NS
K = 128
EPT = 10240
NB = EPT // K
EPAD = EPT * NW
NPAD = 10112
NPT = NPAD // NS
R = 2528
M = 10
P = 5

_BN = 1.0 / (1.001 ** 0.5)
_f32 = jnp.float32


def _make_agg(F, col_out=False):
    @functools.partial(
        pl.kernel,
        out_type=jax.ShapeDtypeStruct((NC, NPAD, 8 if col_out else F), _f32),
        mesh=plsc.VectorSubcoreMesh(core_axis_name="c", subcore_axis_name="s"),
        scratch_types=[
            pltpu.VMEM((NB, K), jnp.int32),
            pltpu.VMEM((NB, K), jnp.int32),
            pltpu.VMEM((M, K, F), _f32),
            pltpu.VMEM_SHARED((NPAD, F), _f32),
            pltpu.SemaphoreType.DMA((M,)),
            pltpu.SemaphoreType.DMA((M,)),
        ],
        compiler_params=pltpu.CompilerParams(use_tc_tiling_on_sc=False),
    )
    def agg(g_hbm, src_hbm, dst_hbm, zero_hbm, out_hbm, src_v, dst_v, buf,
            acc, gsem, ssem):
        cid = lax.axis_index("c")
        sid = lax.axis_index("s")
        wid = cid * NS + sid
        pltpu.sync_copy(src_hbm.at[wid], src_v)
        pltpu.sync_copy(dst_hbm.at[wid], dst_v)

        @pl.when(cid == 0)
        def _():
            pltpu.sync_copy(g_hbm.at[pl.ds(sid * NPT, NPT)],
                            acc.at[pl.ds(sid * NPT, NPT)])

        @pl.when(cid == 1)
        def _():
            pltpu.sync_copy(zero_hbm, acc.at[pl.ds(sid * NPT, NPT)])

        plsc.subcore_barrier()

        for b in range(P):
            pltpu.async_copy(g_hbm.at[src_v.at[b]], buf.at[b], gsem.at[b])

        @pl.loop(0, NB, step=M)
        def _(j):
            for b in range(M):
                t = j + b
                pltpu.make_async_copy(
                    g_hbm.at[src_v.at[t]], buf.at[b], gsem.at[b]).wait()
                pltpu.async_copy(buf.at[b], acc.at[dst_v.at[t]], ssem.at[b],
                                 add=True)
                bg = (b + P) % M

                def _prefetch(bg=bg, t=t):
                    pltpu.make_async_copy(
                        buf.at[bg], acc.at[dst_v.at[t]], ssem.at[bg]).wait()
                    pltpu.async_copy(g_hbm.at[src_v.at[t + P]], buf.at[bg],
                                     gsem.at[bg])

                def _prefetch_first(bg=bg, t=t):
                    pltpu.async_copy(g_hbm.at[src_v.at[t + P]], buf.at[bg],
                                     gsem.at[bg])

                if b + P < M:
                    pl.when(j > 0)(_prefetch)
                    pl.when(jnp.logical_and(j == 0, t + P < NB))(
                        _prefetch_first)
                else:
                    pl.when(t + P < NB)(_prefetch)

        for b in range(M):
            pltpu.make_async_copy(
                buf.at[b], acc.at[dst_v.at[0]], ssem.at[b]).wait()
        plsc.subcore_barrier()
        if col_out:
            pltpu.sync_copy(acc.at[pl.ds(sid * NPT, NPT), pl.ds(0, 8)],
                            out_hbm.at[cid, pl.ds(sid * NPT, NPT)])
        else:
            pltpu.sync_copy(acc.at[pl.ds(sid * NPT, NPT)],
                            out_hbm.at[cid, pl.ds(sid * NPT, NPT)])

    return agg


def _elu(h):
    return jnp.where(h > 0, h, jnp.exp(jnp.minimum(h, 0.0)) - 1.0)


def _rows(shape):
    if len(shape) == 2:
        return pl.BlockSpec((R, shape[1]), lambda i: (i, 0))
    return pl.BlockSpec((shape[0], R, shape[2]), lambda i: (0, i, 0))


def _rep(shape):
    nd = len(shape)
    return pl.BlockSpec(shape, lambda i: (0,) * nd)


def _enc_body(x_ref, pdeg_ref, w1, b1, ga1, be1, w2, b2, ga2, be2,
              wc1, feat_o, g1_o, dinv_o):
    deg = pdeg_ref[0] + pdeg_ref[1]
    dinv = lax.rsqrt(jnp.maximum(deg, 1e-12))
    h = jnp.dot(x_ref[...], w1[...], preferred_element_type=_f32) + b1[...]
    h = _elu(h * (ga1[...] * _BN) + be1[...])
    h = jnp.dot(h, w2[...], preferred_element_type=_f32) + b2[...]
    feat = _elu(h * (ga2[...] * _BN) + be2[...])
    feat_o[...] = feat
    g1_o[...] = jnp.dot(feat, wc1[...], preferred_element_type=_f32) * dinv
    dinv_o[...] = dinv


def _mid_body(p_ref, d_ref, b_ref, w_ref, k_ref, gout_o):
    dp = jnp.dot(d_ref[...], k_ref[...], preferred_element_type=_f32)
    h = jnp.maximum((p_ref[0] + p_ref[1]) * dp + b_ref[...], 0.0)
    gout_o[...] = jnp.dot(h, w_ref[...], preferred_element_type=_f32) * dp


def _midp(p, dinv2, b, w):
    fi = w.shape[0]
    rep = 128 // fi
    rows = NPAD * fi // 128
    pp = p.reshape(NC, rows, 128)
    dr = dinv2.reshape(rows, rep)
    w_ = jnp.kron(jnp.eye(rep, dtype=_f32), w)
    k_ = jnp.kron(jnp.eye(rep, dtype=_f32), jnp.ones((1, fi), _f32))
    b_ = jnp.tile(b, rep).reshape(1, 128)
    rp = NPAD // 16
    return pl.pallas_call(
        _mid_body,
        grid=(rows // rp,),
        in_specs=[pl.BlockSpec((NC, rp, 128), lambda i: (0, i, 0)),
                  pl.BlockSpec((rp, rep), lambda i: (i, 0)),
                  _rep((1, 128)), _rep((128, 128)), _rep((rep, 128))],
        out_specs=pl.BlockSpec((rp, 128), lambda i: (i, 0)),
        out_shape=jax.ShapeDtypeStruct((rows, 128), _f32),
    )(pp, dr, b_, w_, k_)


def _mid4_body(p_ref, d4_ref, d8_ref, b_ref, w_ref, k4_ref, k8_ref, gm_o):
    dp = jnp.dot(d4_ref[...], k4_ref[...], preferred_element_type=_f32)
    h = jnp.maximum((p_ref[0] + p_ref[1]) * dp + b_ref[...], 0.0)
    h3 = h.reshape(h.shape[0] // 2, 2, 128)
    w4 = w_ref[...]
    gm = jnp.concatenate(
        [jnp.dot(h3[:, 0, :], w4, preferred_element_type=_f32),
         jnp.dot(h3[:, 1, :], w4, preferred_element_type=_f32)], axis=-1)
    dp16 = jnp.dot(d8_ref[...], k8_ref[...], preferred_element_type=_f32)
    gm_o[...] = gm * dp16


def _mid4(p, dinv2, b, w):
    pp = p.reshape(NC, NPAD // 4, 128)
    d4 = dinv2.reshape(NPAD // 4, 4)
    d8 = dinv2.reshape(NPAD // 8, 8)
    w4 = jnp.kron(jnp.eye(4, dtype=_f32), w)
    k4 = jnp.kron(jnp.eye(4, dtype=_f32), jnp.ones((1, 32), _f32))
    k8 = jnp.kron(jnp.eye(8, dtype=_f32), jnp.ones((1, 16), _f32))
    b4 = jnp.tile(b, 4).reshape(1, 128)
    rp = NPAD // 8
    return pl.pallas_call(
        _mid4_body,
        grid=(2,),
        in_specs=[pl.BlockSpec((NC, rp, 128), lambda i: (0, i, 0)),
                  pl.BlockSpec((rp, 4), lambda i: (i, 0)),
                  pl.BlockSpec((rp // 2, 8), lambda i: (i, 0)),
                  _rep((1, 128)), _rep((128, 64)), _rep((4, 128)),
                  _rep((8, 128))],
        out_specs=pl.BlockSpec((rp // 2, 128), lambda i: (i, 0)),
        out_shape=jax.ShapeDtypeStruct((NPAD // 8, 128), _f32),
    )(pp, d4, d8, b4, w4, k4, k8)


def _fin_body(p_ref, dinv_ref, b_ref, feat_ref,
              wd1, bd1, ga1, be1, wd2, bd2, ga2, be2, wd3, bd3, ga3, be3,
              ct_ref, z_o, mu_o, lv_o, de_o, q_o):
    dinv = dinv_ref[...]
    h = jnp.maximum((p_ref[0] + p_ref[1]) * dinv + b_ref[...], 0.0)
    mu = h[:, :8]
    lv = h[:, 8:]
    z = jnp.concatenate([feat_ref[...], mu], axis=1)
    d = jnp.dot(z, wd1[...], preferred_element_type=_f32) + bd1[...]
    d = _elu(d * (ga1[...] * _BN) + be1[...])
    d = jnp.dot(d, wd2[...], preferred_element_type=_f32) + bd2[...]
    d = _elu(d * (ga2[...] * _BN) + be2[...])
    d = jnp.dot(d, wd3[...], preferred_element_type=_f32) + bd3[...]
    d = d * (ga3[...] * _BN) + be3[...]
    de = 1.0 / (1.0 + jnp.exp(-d))
    ct = ct_ref[...]
    d2 = (jnp.sum(z * z, axis=1, keepdims=True)
          - 2.0 * jnp.dot(z, ct, preferred_element_type=_f32)
          + jnp.sum(ct * ct, axis=0, keepdims=True))
    qu = 1.0 / ((1.0 + d2 * 1.25) + 1e-8)
    qu = jnp.exp(0.9 * jnp.log(qu))
    z_o[...] = z
    mu_o[...] = mu
    lv_o[...] = lv
    de_o[...] = de
    q_o[...] = qu / jnp.sum(qu, axis=1, keepdims=True)


def kernel(x, edge_index, params):
    pe, pc = params["enc"], params["conv"]
    pm, pv, pd = params["mean"], params["logvar"], params["dec"]

    padi = jnp.asarray(N + np.arange(EPAD - E) % (NPAD - N), jnp.int32)
    srcp = jnp.concatenate([edge_index[0], padi]).reshape(NW, NB, K)
    dstp = jnp.concatenate([edge_index[1], padi]).reshape(NW, NB, K)
    xpad = jnp.concatenate([x, jnp.zeros((NPAD - N, x.shape[1]), _f32)], axis=0)
    ones16 = jnp.ones((NPAD, 16), _f32)
    wml1 = jnp.concatenate([pm[0]["W"], pv[0]["W"]], axis=1)
    z8 = jnp.zeros((8, 8), _f32)
    wblk = jnp.concatenate(
        [jnp.concatenate([pm[1]["W"], z8], axis=1),
         jnp.concatenate([z8, pv[1]["W"]], axis=1)], axis=0)
    bml1 = jnp.concatenate([pm[0]["b"], pv[0]["b"]])
    bml2 = jnp.concatenate([pm[1]["b"], pv[1]["b"]])
    ct = params["cluster"].T

    agg32 = _make_agg(32)
    agg16 = _make_agg(16)
    aggdeg = _make_agg(16, col_out=True)

    zero32 = jnp.zeros((NPT, 32), _f32)
    zero16 = jnp.zeros((NPT, 16), _f32)

    pdeg1 = aggdeg(ones16, dstp, dstp, zero16)[:, :, :1]

    feat, g1, dinv2 = pl.pallas_call(
        _enc_body,
        grid=(NPAD // R,),
        in_specs=[_rows((NPAD, 128)), _rows((NC, NPAD, 1)),
                  _rep((128, 50)), _rep((1, 50)), _rep((1, 50)), _rep((1, 50)),
                  _rep((50, 20)), _rep((1, 20)), _rep((1, 20)), _rep((1, 20)),
                  _rep((20, 32))],
        out_specs=[_rows((NPAD, 20)), _rows((NPAD, 32)), _rows((NPAD, 1))],
        out_shape=[jax.ShapeDtypeStruct((NPAD, 20), _f32),
                   jax.ShapeDtypeStruct((NPAD, 32), _f32),
                   jax.ShapeDtypeStruct((NPAD, 1), _f32)],
    )(xpad, pdeg1,
      pe[0]["W"], pe[0]["b"].reshape(1, 50), pe[0]["g"].reshape(1, 50),
      pe[0]["be"].reshape(1, 50),
      pe[1]["W"], pe[1]["b"].reshape(1, 20), pe[1]["g"].reshape(1, 20),
      pe[1]["be"].reshape(1, 20),
      pc[0]["W"])

    p1 = agg32(g1, srcp, dstp, zero32)
    g2p = _midp(p1, dinv2, pc[0]["b"], pc[1]["W"])
    p2 = agg32(g2p.reshape(NPAD, 32), srcp, dstp, zero32)
    g3p = _midp(p2, dinv2, pc[1]["b"], pc[2]["W"])
    p3 = agg32(g3p.reshape(NPAD, 32), srcp, dstp, zero32)
    g4p = _midp(p3, dinv2, pc[2]["b"], pc[3]["W"])
    p4 = agg32(g4p.reshape(NPAD, 32), srcp, dstp, zero32)
    gmp = _mid4(p4, dinv2, pc[3]["b"], wml1)
    p5 = agg16(gmp.reshape(NPAD, 16), srcp, dstp, zero16)
    gm2p = _midp(p5, dinv2, bml1, wblk)
    p6 = agg16(gm2p.reshape(NPAD, 16), srcp, dstp, zero16)

    z, mu, lv, de, q = pl.pallas_call(
        _fin_body,
        grid=(NPAD // R,),
        in_specs=[_rows((NC, NPAD, 16)), _rows((NPAD, 1)),
                  _rep((1, 16)), _rows((NPAD, 20)),
                  _rep((28, 50)), _rep((1, 50)), _rep((1, 50)), _rep((1, 50)),
                  _rep((50, 60)), _rep((1, 60)), _rep((1, 60)), _rep((1, 60)),
                  _rep((60, 128)), _rep((1, 128)), _rep((1, 128)), _rep((1, 128)),
                  _rep((28, 20))],
        out_specs=[_rows((N, 28)), _rows((N, 8)), _rows((N, 8)),
                   _rows((N, 128)), _rows((N, 20))],
        out_shape=[jax.ShapeDtypeStruct((N, 28), _f32),
                   jax.ShapeDtypeStruct((N, 8), _f32),
                   jax.ShapeDtypeStruct((N, 8), _f32),
                   jax.ShapeDtypeStruct((N, 128), _f32),
                   jax.ShapeDtypeStruct((N, 20), _f32)],
    )(p6, dinv2, bml2.reshape(1, 16), feat,
      pd[0]["W"], pd[0]["b"].reshape(1, 50), pd[0]["g"].reshape(1, 50),
      pd[0]["be"].reshape(1, 50),
      pd[1]["W"], pd[1]["b"].reshape(1, 60), pd[1]["g"].reshape(1, 60),
      pd[1]["be"].reshape(1, 60),
      pd[2]["W"], pd[2]["b"].reshape(1, 128), pd[2]["g"].reshape(1, 128),
      pd[2]["be"].reshape(1, 128),
      ct)

    return z, mu, lv, de, q
